# baseline (speedup 1.0000x reference)
import functools

import jax
import jax.numpy as jnp
from jax import lax
from jax.experimental import pallas as pl
from jax.experimental.pallas import tpu as pltpu
from jax.experimental.pallas import tpu_sc as plsc

_NUM_SINK = 4
_ROWS = 32768
_ROW_W = 1024
_NW = 32
_RPW = _ROWS // _NW
_CH = 8
_NCH = _RPW // _CH
_LANES = 16

_mesh = plsc.VectorSubcoreMesh(core_axis_name="c", subcore_axis_name="s")


@functools.partial(
    pl.kernel,
    out_type=jax.ShapeDtypeStruct((_ROWS, 8, 128), jnp.float32),
    mesh=_mesh,
    compiler_params=pltpu.CompilerParams(needs_layout_passes=False),
    scratch_types=[
        pltpu.VMEM((_LANES,), jnp.int32),
        pltpu.VMEM((_LANES,), jnp.int32),
        pltpu.VMEM((_RPW,), jnp.int32),
        pltpu.VMEM((_CH, 8, 128), jnp.float32),
        pltpu.VMEM((_CH, 8, 128), jnp.float32),
        pltpu.VMEM((_CH, 8, 128), jnp.float32),
        pltpu.SemaphoreType.DMA,
        pltpu.SemaphoreType.DMA,
        pltpu.SemaphoreType.DMA,
        pltpu.SemaphoreType.DMA,
        pltpu.SemaphoreType.DMA,
        pltpu.SemaphoreType.DMA,
        pltpu.VMEM_SHARED((16, 3, _CH, 8, 128), jnp.float32),
        pltpu.SemaphoreType.DMA,
    ],
)
def _evict(cache_hbm, ne_hbm, cl_hbm, out_hbm,
           ne_v, cl_v, idx_all, buf0, buf1, buf2,
           gsem0, gsem1, gsem2, ssem0, ssem1, ssem2, sh, xsem):
    cid = lax.axis_index("c")
    sid = lax.axis_index("s")
    wid = sid * 2 + cid
    b = wid // 2
    base_row = wid * _RPW

    pltpu.sync_copy(ne_hbm, ne_v)
    pltpu.sync_copy(cl_hbm, cl_v)
    lanes = lax.iota(jnp.int32, _LANES)
    bvec = jnp.full((_LANES,), 0, jnp.int32) + b
    ne = plsc.load_gather(ne_v, [bvec])
    tg = plsc.load_gather(cl_v, [bvec]) - ne

    def fill_group(k, carry):
        r = base_row + k * _LANES + lanes
        t = (((r >> 5) - (b << 6)) << 4) + (r & 15)
        st = t + jnp.where((t >= _NUM_SINK) & (t < tg), ne, 0)
        idx_all[pl.ds(k * _LANES, _LANES)] = (
            (b << 11) + ((st >> 4) << 5) + (r & 16) + (st & 15)
        )
        return carry

    lax.fori_loop(0, _RPW // _LANES, fill_group, 0)

    buf = (buf0, buf1, buf2)
    gsem = (gsem0, gsem1, gsem2)
    ssem = (ssem0, ssem1, ssem2)

    def out_slice(c):
        return out_hbm.at[pl.ds(base_row + c * _CH, _CH)]

    def idx_slice(c):
        return idx_all.at[pl.ds(c * _CH, _CH)]

    def gstart(c, j):
        pltpu.async_copy(cache_hbm.at[idx_slice(c)], buf[j], gsem[j])

    gstart(0, 0)

    def do_chunk(c, j, prefetch, drain):
        pltpu.make_async_copy(cache_hbm.at[idx_slice(c)], buf[j], gsem[j]).wait()
        pltpu.async_copy(buf[j], sh.at[sid, j], xsem).wait()
        pltpu.async_copy(sh.at[sid, j], out_slice(c), ssem[j])
        if prefetch:
            j1 = (j + 1) % 3
            if drain:
                pltpu.make_async_copy(sh.at[sid, j1], out_slice(c - 2), ssem[j1]).wait()
            gstart(c + 1, j1)

    def outer(o, carry):
        c = 3 * o
        do_chunk(c, 0, True, True)
        do_chunk(c + 1, 1, True, True)
        do_chunk(c + 2, 2, True, True)
        return carry

    do_chunk(0, 0, True, False)
    do_chunk(1, 1, True, False)
    do_chunk(2, 2, True, True)
    _full = (_NCH - 3) // 3
    lax.fori_loop(1, _full + 1, outer, 0)
    for _c in range(3 * (_full + 1), _NCH):
        do_chunk(_c, _c % 3, _c + 1 < _NCH, True)
    for c in (_NCH - 3, _NCH - 2, _NCH - 1):
        pltpu.make_async_copy(sh.at[sid, c % 3], out_slice(c), ssem[c % 3]).wait()


def kernel(kv_cache, num_evicts, cachelens, n_local_heads, head_dim):
    flat = kv_cache.reshape(_ROWS, 8, 128)
    out = _evict(flat,
                 num_evicts.astype(jnp.int32),
                 cachelens.astype(jnp.int32))
    return out.reshape(kv_cache.shape)


# speedup vs baseline: 1.2082x; 1.2082x over previous
import functools

import jax
import jax.numpy as jnp
from jax import lax
from jax.experimental import pallas as pl
from jax.experimental.pallas import tpu as pltpu
from jax.experimental.pallas import tpu_sc as plsc

_NUM_SINK = 4
_ROWS = 32768
_ROW_W = 1024
_NW = 32
_RPW = _ROWS // _NW
_CH = 8
_NCH = _RPW // _CH
_LANES = 16

_mesh = plsc.VectorSubcoreMesh(core_axis_name="c", subcore_axis_name="s")


@functools.partial(
    pl.kernel,
    out_type=jax.ShapeDtypeStruct((_ROWS, 8, 128), jnp.float32),
    mesh=_mesh,
    compiler_params=pltpu.CompilerParams(needs_layout_passes=False),
    scratch_types=[
        pltpu.VMEM((_LANES,), jnp.int32),
        pltpu.VMEM((_LANES,), jnp.int32),
        pltpu.VMEM((_RPW,), jnp.int32),
        pltpu.VMEM((_CH, 8, 128), jnp.float32),
        pltpu.VMEM((_CH, 8, 128), jnp.float32),
        pltpu.VMEM((_CH, 8, 128), jnp.float32),
        pltpu.SemaphoreType.DMA,
        pltpu.SemaphoreType.DMA,
        pltpu.SemaphoreType.DMA,
        pltpu.SemaphoreType.DMA,
        pltpu.SemaphoreType.DMA,
        pltpu.SemaphoreType.DMA,
    ],
)
def _evict(cache_hbm, ne_hbm, cl_hbm, out_hbm,
           ne_v, cl_v, idx_all, buf0, buf1, buf2,
           gsem0, gsem1, gsem2, ssem0, ssem1, ssem2):
    cid = lax.axis_index("c")
    sid = lax.axis_index("s")
    wid = sid * 2 + cid
    b = wid // 2
    base_row = wid * _RPW

    pltpu.sync_copy(ne_hbm, ne_v)
    pltpu.sync_copy(cl_hbm, cl_v)
    lanes = lax.iota(jnp.int32, _LANES)
    bvec = jnp.full((_LANES,), 0, jnp.int32) + b
    ne = plsc.load_gather(ne_v, [bvec])
    tg = plsc.load_gather(cl_v, [bvec]) - ne

    def fill_group(k, carry):
        r = base_row + k * _LANES + lanes
        t = (((r >> 5) - (b << 6)) << 4) + (r & 15)
        st = t + jnp.where((t >= _NUM_SINK) & (t < tg), ne, 0)
        idx_all[pl.ds(k * _LANES, _LANES)] = (
            (b << 11) + ((st >> 4) << 5) + (r & 16) + (st & 15)
        )
        return carry

    lax.fori_loop(0, _RPW // _LANES, fill_group, 0)

    buf = (buf0, buf1, buf2)
    gsem = (gsem0, gsem1, gsem2)
    ssem = (ssem0, ssem1, ssem2)

    def out_slice(c):
        return out_hbm.at[pl.ds(base_row + c * _CH, _CH)]

    def idx_slice(c):
        return idx_all.at[pl.ds(c * _CH, _CH)]

    def gstart(c, j):
        pltpu.async_copy(cache_hbm.at[idx_slice(c)], buf[j], gsem[j])

    gstart(0, 0)

    def do_chunk(c, j, prefetch, drain):
        pltpu.make_async_copy(cache_hbm.at[idx_slice(c)], buf[j], gsem[j]).wait()
        pltpu.async_copy(buf[j], out_slice(c), ssem[j])
        if prefetch:
            j1 = (j + 1) % 3
            if drain:
                pltpu.make_async_copy(buf[j1], out_slice(c - 2), ssem[j1]).wait()
            gstart(c + 1, j1)

    def outer(o, carry):
        c = 3 * o
        do_chunk(c, 0, True, True)
        do_chunk(c + 1, 1, True, True)
        do_chunk(c + 2, 2, True, True)
        return carry

    do_chunk(0, 0, True, False)
    do_chunk(1, 1, True, False)
    do_chunk(2, 2, True, True)
    _full = (_NCH - 3) // 3
    lax.fori_loop(1, _full + 1, outer, 0)
    for _c in range(3 * (_full + 1), _NCH):
        do_chunk(_c, _c % 3, _c + 1 < _NCH, True)
    for c in (_NCH - 3, _NCH - 2, _NCH - 1):
        pltpu.make_async_copy(buf[c % 3], out_slice(c), ssem[c % 3]).wait()


def kernel(kv_cache, num_evicts, cachelens, n_local_heads, head_dim):
    flat = kv_cache.reshape(_ROWS, 8, 128)
    out = _evict(flat,
                 num_evicts.astype(jnp.int32),
                 cachelens.astype(jnp.int32))
    return out.reshape(kv_cache.shape)


# speedup vs baseline: 1.9143x; 1.5844x over previous
import functools

import jax
import jax.numpy as jnp
from jax import lax
from jax.experimental import pallas as pl
from jax.experimental.pallas import tpu as pltpu
from jax.experimental.pallas import tpu_sc as plsc

_NUM_SINK = 4
_ROWS = 32768
_NW = 32
_RPW = _ROWS // _NW
_CH = 56
_NFULL = _RPW // _CH
_CHT = _RPW - _NFULL * _CH
_LANES = 16

_mesh = plsc.VectorSubcoreMesh(core_axis_name="c", subcore_axis_name="s")


@functools.partial(
    pl.kernel,
    out_type=jax.ShapeDtypeStruct((_ROWS, 8, 128), jnp.float32),
    mesh=_mesh,
    compiler_params=pltpu.CompilerParams(needs_layout_passes=False),
    scratch_types=[
        pltpu.VMEM((_LANES,), jnp.int32),
        pltpu.VMEM((_LANES,), jnp.int32),
        pltpu.VMEM((_RPW,), jnp.int32),
        pltpu.VMEM((_CH, 8, 128), jnp.float32),
        pltpu.VMEM((_CH, 8, 128), jnp.float32),
        pltpu.SemaphoreType.DMA,
        pltpu.SemaphoreType.DMA,
        pltpu.SemaphoreType.DMA,
        pltpu.SemaphoreType.DMA,
    ],
)
def _evict(cache_hbm, ne_hbm, cl_hbm, out_hbm,
           ne_v, cl_v, idx_all, buf0, buf1,
           gsem0, gsem1, ssem0, ssem1):
    cid = lax.axis_index("c")
    sid = lax.axis_index("s")
    wid = sid * 2 + cid
    b = wid // 2
    base_row = wid * _RPW

    pltpu.sync_copy(ne_hbm, ne_v)
    pltpu.sync_copy(cl_hbm, cl_v)
    lanes = lax.iota(jnp.int32, _LANES)
    bvec = jnp.full((_LANES,), 0, jnp.int32) + b
    ne = plsc.load_gather(ne_v, [bvec])
    tg = plsc.load_gather(cl_v, [bvec]) - ne

    def fill_group(k, carry):
        r = base_row + k * _LANES + lanes
        t = (((r >> 5) - (b << 6)) << 4) + (r & 15)
        st = t + jnp.where((t >= _NUM_SINK) & (t < tg), ne, 0)
        idx_all[pl.ds(k * _LANES, _LANES)] = (
            (b << 11) + ((st >> 4) << 5) + (r & 16) + (st & 15)
        )
        return carry

    lax.fori_loop(0, _RPW // _LANES, fill_group, 0)

    buf = (buf0, buf1)
    gsem = (gsem0, gsem1)
    ssem = (ssem0, ssem1)

    def bref(j, n):
        return buf[j] if n == _CH else buf[j].at[pl.ds(0, n)]

    def gather(c, j, n):
        return pltpu.make_async_copy(
            cache_hbm.at[idx_all.at[pl.ds(c * _CH, n)]], bref(j, n), gsem[j])

    def scatter(c, j, n):
        return pltpu.make_async_copy(
            bref(j, n), out_hbm.at[pl.ds(base_row + c * _CH, n)], ssem[j])

    def do_chunk(c, j, n, pn, drain):
        gather(c, j, n).wait()
        scatter(c, j, n).start()
        if pn:
            if drain:
                scatter(c - 1, 1 - j, _CH).wait()
            gather(c + 1, 1 - j, pn).start()

    gather(0, 0, _CH).start()
    do_chunk(0, 0, _CH, _CH, False)

    def outer(o, carry):
        do_chunk(2 * o + 1, 1, _CH, _CH, True)
        do_chunk(2 * o + 2, 0, _CH, _CH, True)
        return carry

    lax.fori_loop(0, (_NFULL - 2) // 2, outer, 0)
    do_chunk(_NFULL - 1, 1, _CH, _CHT, True)
    do_chunk(_NFULL, 0, _CHT, 0, False)
    scatter(_NFULL - 1, 1, _CH).wait()
    scatter(_NFULL, 0, _CHT).wait()


def kernel(kv_cache, num_evicts, cachelens, n_local_heads, head_dim):
    flat = kv_cache.reshape(_ROWS, 8, 128)
    out = _evict(flat,
                 num_evicts.astype(jnp.int32),
                 cachelens.astype(jnp.int32))
    return out.reshape(kv_cache.shape)


# speedup vs baseline: 1.9171x; 1.0015x over previous
import functools

import jax
import jax.numpy as jnp
from jax import lax
from jax.experimental import pallas as pl
from jax.experimental.pallas import tpu as pltpu
from jax.experimental.pallas import tpu_sc as plsc

_NUM_SINK = 4
_ROWS = 32768
_NW = 32
_RPW = _ROWS // _NW
_CH = 56
_NFULL = _RPW // _CH
_CHT = _RPW - _NFULL * _CH
_LANES = 16

@functools.cache
def _make_evict():
    mesh = plsc.VectorSubcoreMesh(core_axis_name="c", subcore_axis_name="s")
    return functools.partial(
        pl.kernel,
        out_type=jax.ShapeDtypeStruct((_ROWS, 8, 128), jnp.float32),
        mesh=mesh,
        compiler_params=pltpu.CompilerParams(needs_layout_passes=False),
        scratch_types=[
            pltpu.VMEM((_LANES,), jnp.int32),
            pltpu.VMEM((_LANES,), jnp.int32),
            pltpu.VMEM((_RPW,), jnp.int32),
            pltpu.VMEM((_CH, 8, 128), jnp.float32),
            pltpu.VMEM((_CH, 8, 128), jnp.float32),
            pltpu.SemaphoreType.DMA,
            pltpu.SemaphoreType.DMA,
            pltpu.SemaphoreType.DMA,
            pltpu.SemaphoreType.DMA,
        ],
    )(_evict)


def _evict(cache_hbm, ne_hbm, cl_hbm, out_hbm,
           ne_v, cl_v, idx_all, buf0, buf1,
           gsem0, gsem1, ssem0, ssem1):
    cid = lax.axis_index("c")
    sid = lax.axis_index("s")
    wid = sid * 2 + cid
    b = wid // 2
    base_row = wid * _RPW

    pltpu.sync_copy(ne_hbm, ne_v)
    pltpu.sync_copy(cl_hbm, cl_v)
    lanes = lax.iota(jnp.int32, _LANES)
    bvec = jnp.full((_LANES,), 0, jnp.int32) + b
    ne = plsc.load_gather(ne_v, [bvec])
    tg = plsc.load_gather(cl_v, [bvec]) - ne

    def fill_group(k, carry):
        r = base_row + k * _LANES + lanes
        t = (((r >> 5) - (b << 6)) << 4) + (r & 15)
        st = t + jnp.where((t >= _NUM_SINK) & (t < tg), ne, 0)
        idx_all[pl.ds(k * _LANES, _LANES)] = (
            (b << 11) + ((st >> 4) << 5) + (r & 16) + (st & 15)
        )
        return carry

    lax.fori_loop(0, _RPW // _LANES, fill_group, 0)

    buf = (buf0, buf1)
    gsem = (gsem0, gsem1)
    ssem = (ssem0, ssem1)

    def bref(j, n):
        return buf[j] if n == _CH else buf[j].at[pl.ds(0, n)]

    def gather(c, j, n):
        return pltpu.make_async_copy(
            cache_hbm.at[idx_all.at[pl.ds(c * _CH, n)]], bref(j, n), gsem[j])

    def scatter(c, j, n):
        return pltpu.make_async_copy(
            bref(j, n), out_hbm.at[pl.ds(base_row + c * _CH, n)], ssem[j])

    def do_chunk(c, j, n, pn, drain):
        gather(c, j, n).wait()
        scatter(c, j, n).start()
        if pn:
            if drain:
                scatter(c - 1, 1 - j, _CH).wait()
            gather(c + 1, 1 - j, pn).start()

    gather(0, 0, _CH).start()
    do_chunk(0, 0, _CH, _CH, False)

    def outer(o, carry):
        do_chunk(2 * o + 1, 1, _CH, _CH, True)
        do_chunk(2 * o + 2, 0, _CH, _CH, True)
        return carry

    lax.fori_loop(0, (_NFULL - 2) // 2, outer, 0)
    do_chunk(_NFULL - 1, 1, _CH, _CHT, True)
    do_chunk(_NFULL, 0, _CHT, 0, False)
    scatter(_NFULL - 1, 1, _CH).wait()
    scatter(_NFULL, 0, _CHT).wait()


def kernel(kv_cache, num_evicts, cachelens, n_local_heads, head_dim):
    flat = kv_cache.reshape(_ROWS, 8, 128)
    out = _make_evict()(flat,
                        num_evicts.astype(jnp.int32),
                        cachelens.astype(jnp.int32))
    return out.reshape(kv_cache.shape)
